# baseline (speedup 1.0000x reference)
import functools

import jax
import jax.numpy as jnp
from jax import lax
from jax.experimental import pallas as pl
from jax.experimental.pallas import tpu as pltpu

_NBUF = 4


def _fused_kernel(x_ref, w_ref, adj_ref, out_ref, h_ref, bufs_ref, sems, *,
                  bm, steps):
    h_ref[...] = lax.dot_general(
        x_ref[...], w_ref[...],
        (((1,), (1,)), ((), ())),
        preferred_element_type=jnp.float32)

    def _copy(j, slot):
        return pltpu.make_async_copy(
            adj_ref.at[pl.ds(j * bm, bm), :],
            bufs_ref.at[slot],
            sems.at[slot])

    for j in range(_NBUF):
        _copy(j, j).start()

    def _body(j, carry):
        slot = lax.rem(j, _NBUF)
        _copy(j, slot).wait()
        out_ref[pl.ds(j * bm, bm), :] = jnp.dot(
            bufs_ref[slot], h_ref[...], preferred_element_type=jnp.float32)

        @pl.when(j + _NBUF < steps)
        def _():
            _copy(j + _NBUF, slot).start()

        return carry

    lax.fori_loop(0, steps, _body, 0)


def kernel(x, adj, W):
    n, d_in = x.shape
    d_out = W.shape[0]

    bm = 200
    steps = n // bm
    return pl.pallas_call(
        functools.partial(_fused_kernel, bm=bm, steps=steps),
        in_specs=[
            pl.BlockSpec(memory_space=pltpu.VMEM),
            pl.BlockSpec(memory_space=pltpu.VMEM),
            pl.BlockSpec(memory_space=pl.ANY),
        ],
        out_specs=pl.BlockSpec(memory_space=pltpu.VMEM),
        out_shape=jax.ShapeDtypeStruct((n, d_out), jnp.float32),
        scratch_shapes=[
            pltpu.VMEM((n, d_out), jnp.float32),
            pltpu.VMEM((_NBUF, bm, n), jnp.float32),
            pltpu.SemaphoreType.DMA((_NBUF,)),
        ],
    )(x, W, adj)


# speedup vs baseline: 1.0035x; 1.0035x over previous
import functools

import jax
import jax.numpy as jnp
from jax import lax
from jax.experimental import pallas as pl
from jax.experimental.pallas import tpu as pltpu

_NBUF = 4


def _fused_kernel(x_ref, w_ref, adj_ref, out_ref, h_ref, bufs_ref, sems, *,
                  bm, steps):
    def _copy(j, slot):
        return pltpu.make_async_copy(
            adj_ref.at[pl.ds(j * bm, bm), :],
            bufs_ref.at[slot],
            sems.at[slot])

    for j in range(_NBUF):
        _copy(j, j).start()

    h_ref[...] = lax.dot_general(
        x_ref[...], w_ref[...],
        (((1,), (1,)), ((), ())),
        preferred_element_type=jnp.float32)

    def _body(j, carry):
        slot = lax.rem(j, _NBUF)
        _copy(j, slot).wait()
        out_ref[pl.ds(j * bm, bm), :] = jnp.dot(
            bufs_ref[slot], h_ref[...], preferred_element_type=jnp.float32)

        @pl.when(j + _NBUF < steps)
        def _():
            _copy(j + _NBUF, slot).start()

        return carry

    lax.fori_loop(0, steps, _body, 0)


def kernel(x, adj, W):
    n, d_in = x.shape
    d_out = W.shape[0]

    bm = 200
    steps = n // bm
    return pl.pallas_call(
        functools.partial(_fused_kernel, bm=bm, steps=steps),
        in_specs=[
            pl.BlockSpec(memory_space=pltpu.VMEM),
            pl.BlockSpec(memory_space=pltpu.VMEM),
            pl.BlockSpec(memory_space=pl.ANY),
        ],
        out_specs=pl.BlockSpec(memory_space=pltpu.VMEM),
        out_shape=jax.ShapeDtypeStruct((n, d_out), jnp.float32),
        scratch_shapes=[
            pltpu.VMEM((n, d_out), jnp.float32),
            pltpu.VMEM((_NBUF, bm, n), jnp.float32),
            pltpu.SemaphoreType.DMA((_NBUF,)),
        ],
    )(x, W, adj)


# speedup vs baseline: 1.0178x; 1.0143x over previous
import jax
import jax.numpy as jnp
from jax import lax
from jax.experimental import pallas as pl
from jax.experimental.pallas import tpu as pltpu


def _fused_kernel(x_ref, w_ref, adj_ref, out_ref, h_ref):
    @pl.when(pl.program_id(0) == 0)
    def _():
        h_ref[...] = lax.dot_general(
            x_ref[...], w_ref[...],
            (((1,), (1,)), ((), ())),
            preferred_element_type=jnp.float32)

    out_ref[...] = jnp.dot(adj_ref[...], h_ref[...],
                           preferred_element_type=jnp.float32)


def kernel(x, adj, W):
    n, d_in = x.shape
    d_out = W.shape[0]

    bm = 400
    return pl.pallas_call(
        _fused_kernel,
        grid=(n // bm,),
        in_specs=[
            pl.BlockSpec((n, d_in), lambda i: (0, 0)),
            pl.BlockSpec((d_out, d_in), lambda i: (0, 0)),
            pl.BlockSpec((bm, n), lambda i: (i, 0)),
        ],
        out_specs=pl.BlockSpec((bm, d_out), lambda i: (i, 0)),
        out_shape=jax.ShapeDtypeStruct((n, d_out), jnp.float32),
        scratch_shapes=[pltpu.VMEM((n, d_out), jnp.float32)],
        compiler_params=pltpu.CompilerParams(
            dimension_semantics=("arbitrary",),
        ),
    )(x, W, adj)
